# Initial kernel scaffold; baseline (speedup 1.0000x reference)
import jax
import jax.numpy as jnp
from jax.experimental import pallas as pl


def kernel(nfeat, atom_emb, efeat, A_lin1, A_lin2, e1_W, e1_b, e1_attn, n1_W, n1_b, e2_W, e2_b, e2_attn, e2_Wih, e2_Whh, n2_W, n2_b, n2_Wih, n2_Whh, e3_W, e3_b, e3_attn, e3_Wih, e3_Whh, n3_W, n3_b, n3_Wih, n3_Whh, out_W, out_b, edge_index):
    raise NotImplementedError("write your pallas kernel here")


# speedup vs baseline: 4.3280x; 4.3280x over previous
import functools
import jax
import jax.numpy as jnp
from jax import lax
from jax.experimental import pallas as pl
from jax.experimental.pallas import tpu as pltpu
from jax.experimental.pallas import tpu_sc as plsc

N = 50000
E = 800000
EPS = 1e-5
F32 = jnp.float32

EB = 3200
NB = 2000
EXTW = 80

G_SUB = 128
G_NSUB = 8
G_CHUNK = G_SUB * G_NSUB
EPAD = -(-E // G_CHUNK) * G_CHUNK
G_NCHUNK = EPAD // G_CHUNK
G_KMAX = (G_NCHUNK + 31) // 32

S_SUB = 128
S_NSUB = 2
S_CHUNK = S_SUB * S_NSUB
S_NCHUNK = E // S_CHUNK
S_KMAX = (S_NCHUNK + 15) // 16
HALF = N // 2
ACC_ROWS = HALF + 8
ZROWS = 1568


def _rowspec(b, d):
    return pl.BlockSpec((b, d), lambda i: (i, 0))


def _full(shape):
    return pl.BlockSpec(shape, lambda i: tuple(0 for _ in shape))


def _dot(a, b):
    return jnp.dot(a, b, preferred_element_type=F32)


def _atom_body(nf_r, ae_r, a1t_r, a2t_r, out_r):
    ae = ae_r[...]
    y = None
    for a in range(14):
        xa = ae[:, 7 * a:7 * a + 7]
        m = jnp.mean(xa, axis=1, keepdims=True)
        v = jnp.mean(xa * xa, axis=1, keepdims=True) - m * m
        na = (xa - m) * lax.rsqrt(v + EPS)
        y = na if y is None else y + na
    y = y * (1.0 / 14.0)
    t = jnp.maximum(_dot(y, a1t_r[...]), 0.0)
    m2 = jnp.mean(t, axis=1, keepdims=True)
    v2 = jnp.mean(t * t, axis=1, keepdims=True) - m2 * m2
    t = (t - m2) * lax.rsqrt(v2 + EPS)
    t = jnp.maximum(_dot(t, a2t_r[...]), 0.0)
    z4 = jnp.zeros((t.shape[0], 4), F32)
    out_r[...] = jnp.concatenate([nf_r[...], t, z4], axis=1)


def _atom_call(nfeat, ae98, a1t, a2t):
    grid = (N // NB,)
    return pl.pallas_call(
        _atom_body,
        grid=grid,
        in_specs=[_rowspec(NB, 28), _rowspec(NB, 98), _full((7, 64)), _full((64, 64))],
        out_specs=_rowspec(NB, 96),
        out_shape=jax.ShapeDtypeStruct((N, 96), F32),
    )(nfeat, ae98, a1t, a2t)


def _make_edge_body(n_tot, lstm, has_c0, head):
    def body(*refs):
        i = 0
        hs_r = refs[i]; i += 1
        hd_r = refs[i]; i += 1
        e_r = refs[i]; i += 1
        c0_r = None
        if has_c0:
            c0_r = refs[i]; i += 1
        wst_r = refs[i]; i += 1
        wet_r = refs[i]; i += 1
        wdt_r = refs[i]; i += 1
        rs_r = refs[i]; i += 1
        b_r = refs[i]; i += 1
        wih_r = whh_r = None
        if lstm:
            wih_r = refs[i]; i += 1
            whh_r = refs[i]; i += 1
        if head:
            owt_r = refs[i]; i += 1
            ob_r = refs[i]; i += 1
            head_r = refs[i]; i += 1
        else:
            wa_r = refs[i]; i += 1
            eo_r = refs[i]; i += 1
            if lstm:
                co_r = refs[i]; i += 1
            exta_r = refs[i]; i += 1
            extb_r = refs[i]; i += 1

        hs = hs_r[...]
        hd = hd_r[...]
        e = e_r[...]
        s1 = (jnp.sum(hs, 1, keepdims=True) + jnp.sum(e, 1, keepdims=True)
              + jnp.sum(hd, 1, keepdims=True))
        s2 = (jnp.sum(hs * hs, 1, keepdims=True) + jnp.sum(e * e, 1, keepdims=True)
              + jnp.sum(hd * hd, 1, keepdims=True))
        m = s1 * (1.0 / n_tot)
        var = s2 * (1.0 / n_tot) - m * m
        rsig = lax.rsqrt(var + EPS)
        z = _dot(hs, wst_r[...]) + _dot(e, wet_r[...]) + _dot(hd, wdt_r[...])
        x = (z - m * rs_r[...]) * rsig + b_r[...]
        x = jnp.where(x >= 0, x, 0.01 * x)
        if lstm:
            gates = _dot(x, wih_r[...]) + _dot(e, whh_r[...])
            gi = jax.nn.sigmoid(gates[:, 0:64])
            gg = jnp.tanh(gates[:, 128:192])
            go = jax.nn.sigmoid(gates[:, 192:256])
            if has_c0:
                gf = jax.nn.sigmoid(gates[:, 64:128])
                c = gf * c0_r[...] + gi * gg
            else:
                c = gi * gg
            x = go * jnp.tanh(c)
        if head:
            head_r[...] = _dot(x, owt_r[...]) + ob_r[...]
            return
        eo_r[...] = x
        if lstm:
            co_r[...] = c
        attn = jnp.sum(x * wa_r[...], axis=1, keepdims=True)
        p = jnp.exp(attn)
        xp = x * p
        z15 = jnp.zeros((x.shape[0], 15), F32)
        exta_r[...] = xp[:, 0:48]
        extb_r[...] = jnp.concatenate([xp[:, 48:64], p, z15], axis=1)
    return body


def _edge_call(hs, hd, e, c0, W, b, wa, wih, whh, owt, ob, dh, dh_valid, de):
    lstm = wih is not None
    has_c0 = c0 is not None
    head = owt is not None
    n_tot = 2 * dh_valid + de
    wst = jnp.zeros((dh, 64), F32).at[:dh_valid].set(W[:, :dh_valid].T)
    wet = W[:, dh_valid:dh_valid + de].T
    wdt = jnp.zeros((dh, 64), F32).at[:dh_valid].set(W[:, dh_valid + de:].T)
    rs = W.sum(1)[None, :]
    b2 = b[None, :]
    grid = (E // EB,)
    args = [hs, hd, e]
    in_specs = [_rowspec(EB, dh), _rowspec(EB, dh), _rowspec(EB, de)]
    if has_c0:
        args.append(c0)
        in_specs.append(_rowspec(EB, 64))
    args += [wst, wet, wdt, rs, b2]
    in_specs += [_full((dh, 64)), _full((de, 64)), _full((dh, 64)),
                 _full((1, 64)), _full((1, 64))]
    if lstm:
        args += [wih.T, whh.T]
        in_specs += [_full((64, 256)), _full((64, 256))]
    if head:
        args += [owt.T, ob[None, :]]
        in_specs += [_full((64, 37)), _full((1, 37))]
        out_specs = _rowspec(EB, 37)
        out_shape = jax.ShapeDtypeStruct((E, 37), F32)
    else:
        args += [wa]
        in_specs += [_full((1, 64))]
        out_specs = [_rowspec(EB, 64)]
        out_shape = [jax.ShapeDtypeStruct((E, 64), F32)]
        if lstm:
            out_specs.append(_rowspec(EB, 64))
            out_shape.append(jax.ShapeDtypeStruct((E, 64), F32))
        out_specs.append(_rowspec(EB, 48))
        out_shape.append(jax.ShapeDtypeStruct((E, 48), F32))
        out_specs.append(_rowspec(EB, 32))
        out_shape.append(jax.ShapeDtypeStruct((E, 32), F32))
    body = _make_edge_body(n_tot, lstm, has_c0, head)
    return pl.pallas_call(
        body, grid=grid, in_specs=in_specs, out_specs=out_specs,
        out_shape=out_shape,
    )(*args)


def _make_node_body(n_tot, lstm, has_c0):
    def body(*refs):
        i = 0
        h_r = refs[i]; i += 1
        tsa_r = refs[i]; i += 1
        tsb_r = refs[i]; i += 1
        c0_r = None
        if has_c0:
            c0_r = refs[i]; i += 1
        wht_r = refs[i]; i += 1
        wat_r = refs[i]; i += 1
        rs_r = refs[i]; i += 1
        b_r = refs[i]; i += 1
        if lstm:
            wih_r = refs[i]; i += 1
            whh_r = refs[i]; i += 1
        ho_r = refs[i]; i += 1
        if lstm:
            co_r = refs[i]; i += 1

        h = h_r[...]
        tsa = tsa_r[...]
        tsb = tsb_r[...]
        s = tsb[:, 16:17]
        sinv = jnp.where(s > 0, 1.0 / s, 0.0)
        af = jnp.concatenate([tsa, tsb[:, 0:16]], axis=1) * sinv
        s1 = jnp.sum(h, 1, keepdims=True) + jnp.sum(af, 1, keepdims=True)
        s2 = jnp.sum(h * h, 1, keepdims=True) + jnp.sum(af * af, 1, keepdims=True)
        m = s1 * (1.0 / n_tot)
        var = s2 * (1.0 / n_tot) - m * m
        rsig = lax.rsqrt(var + EPS)
        z = _dot(h, wht_r[...]) + _dot(af, wat_r[...])
        x = (z - m * rs_r[...]) * rsig + b_r[...]
        x = jnp.maximum(x, 0.0)
        if lstm:
            gates = _dot(x, wih_r[...]) + _dot(h, whh_r[...])
            gi = jax.nn.sigmoid(gates[:, 0:64])
            gg = jnp.tanh(gates[:, 128:192])
            go = jax.nn.sigmoid(gates[:, 192:256])
            if has_c0:
                gf = jax.nn.sigmoid(gates[:, 64:128])
                c = gf * c0_r[...] + gi * gg
            else:
                c = gi * gg
            ho_r[...] = go * jnp.tanh(c)
            co_r[...] = c
        else:
            ho_r[...] = x
    return body


def _node_call(h, tsa, tsb, c0, W, b, wih, whh, dh, dh_valid):
    lstm = wih is not None
    has_c0 = c0 is not None
    n_tot = dh_valid + 64
    wht = jnp.zeros((dh, 64), F32).at[:dh_valid].set(W[:, :dh_valid].T)
    wat = W[:, dh_valid:].T
    rs = W.sum(1)[None, :]
    b2 = b[None, :]
    grid = (N // NB,)
    args = [h, tsa, tsb]
    in_specs = [_rowspec(NB, dh), _rowspec(NB, 48), _rowspec(NB, 32)]
    if has_c0:
        args.append(c0)
        in_specs.append(_rowspec(NB, 64))
    args += [wht, wat, rs, b2]
    in_specs += [_full((dh, 64)), _full((64, 64)), _full((1, 64)), _full((1, 64))]
    if lstm:
        args += [wih.T, whh.T]
        in_specs += [_full((64, 256)), _full((64, 256))]
        out_specs = [_rowspec(NB, 64), _rowspec(NB, 64)]
        out_shape = [jax.ShapeDtypeStruct((N, 64), F32),
                     jax.ShapeDtypeStruct((N, 64), F32)]
    else:
        out_specs = _rowspec(NB, 64)
        out_shape = jax.ShapeDtypeStruct((N, 64), F32)
    body = _make_node_body(n_tot, lstm, has_c0)
    return pl.pallas_call(
        body, grid=grid, in_specs=in_specs, out_specs=out_specs,
        out_shape=out_shape,
    )(*args)


def _make_gather(D):
    mesh = plsc.VectorSubcoreMesh(core_axis_name="c", subcore_axis_name="s")

    @functools.partial(
        pl.kernel, mesh=mesh,
        compiler_params=pltpu.CompilerParams(use_tc_tiling_on_sc=False),
        out_type=[jax.ShapeDtypeStruct((EPAD, D), F32),
                  jax.ShapeDtypeStruct((EPAD, D), F32)],
        scratch_types=[pltpu.VMEM((G_NSUB, G_SUB), jnp.int32),
                       pltpu.VMEM((G_CHUNK, D), F32),
                       pltpu.SemaphoreType.DMA],
    )
    def g(h_hbm, src_hbm, dst_hbm, hs_hbm, hd_hbm, idx_v, rows_v, sem):
        wid = lax.axis_index("s") * 2 + lax.axis_index("c")

        def one(idx2_hbm, out_hbm, j):
            base = j * G_CHUNK
            pltpu.sync_copy(idx2_hbm.at[pl.ds(j * G_NSUB, G_NSUB)], idx_v)
            cps = [pltpu.async_copy(h_hbm.at[idx_v.at[k]],
                                    rows_v.at[pl.ds(k * G_SUB, G_SUB)], sem)
                   for k in range(G_NSUB)]
            for cp in cps:
                cp.wait()
            pltpu.sync_copy(rows_v, out_hbm.at[pl.ds(base, G_CHUNK)])

        def step(k, carry):
            j = wid + k * 32

            @pl.when(j < G_NCHUNK)
            def _():
                one(src_hbm, hs_hbm, j)
                one(dst_hbm, hd_hbm, j)
            return carry

        lax.fori_loop(0, G_KMAX, step, 0)

    return g


_gather96 = _make_gather(96)
_gather64 = _make_gather(64)


def _make_scatter(W):
    mesh = plsc.VectorSubcoreMesh(core_axis_name="c", subcore_axis_name="s")

    @functools.partial(
        pl.kernel, mesh=mesh,
        compiler_params=pltpu.CompilerParams(use_tc_tiling_on_sc=False),
        out_type=jax.ShapeDtypeStruct((N, W), F32),
        scratch_types=[pltpu.VMEM((S_CHUNK,), jnp.int32),
                       pltpu.VMEM((S_SUB,), jnp.int32),
                       pltpu.VMEM((S_SUB,), jnp.int32),
                       pltpu.VMEM((S_CHUNK, W), F32),
                       pltpu.VMEM_SHARED((ACC_ROWS, W), F32)],
    )
    def s(ext_hbm, dst1_hbm, zeros_hbm, out_hbm, idx1_v, idx2a_v, idx2b_v, rows_v, acc):
        cid = lax.axis_index("c")
        sid = lax.axis_index("s")
        node_base = cid * HALF

        pltpu.sync_copy(zeros_hbm, rows_v.at[pl.ds(0, 224)])

        def zslab(r, carry):
            slab = sid + r * 16

            @pl.when(slab < 111)
            def _():
                pltpu.sync_copy(rows_v.at[pl.ds(0, 224)],
                                acc.at[pl.ds(slab * 224, 224)])

            @pl.when(slab == 111)
            def _():
                pltpu.sync_copy(rows_v.at[pl.ds(0, 144)],
                                acc.at[pl.ds(slab * 224, 144)])
            return carry

        lax.fori_loop(0, 7, zslab, 0)
        plsc.subcore_barrier()

        def step(k, carry):
            j = sid + k * 16

            @pl.when(j < S_NCHUNK)
            def _():
                ebase = j * S_CHUNK
                pltpu.sync_copy(dst1_hbm.at[pl.ds(ebase, S_CHUNK)], idx1_v)
                pltpu.sync_copy(ext_hbm.at[pl.ds(ebase, S_CHUNK)], rows_v)
                for g in range(S_CHUNK // 16):
                    d = idx1_v[pl.ds(g * 16, 16)]
                    local = d - node_base
                    ok = (local >= 0) & (local < HALF)
                    trash = HALF + (d & 7)
                    idx2 = jnp.where(ok, local, trash)
                    tgt = idx2a_v if g < 8 else idx2b_v
                    tgt[pl.ds((g % 8) * 16, 16)] = idx2
                pltpu.sync_copy(rows_v.at[pl.ds(0, S_SUB)],
                                acc.at[idx2a_v], add=True)
                pltpu.sync_copy(rows_v.at[pl.ds(S_SUB, S_SUB)],
                                acc.at[idx2b_v], add=True)
            return carry

        lax.fori_loop(0, S_KMAX, step, 0)
        plsc.subcore_barrier()

        def wslab(r, carry):
            slab = sid + r * 16

            @pl.when(slab < 111)
            def _():
                pltpu.sync_copy(acc.at[pl.ds(slab * 224, 224)],
                                rows_v.at[pl.ds(0, 224)])
                pltpu.sync_copy(rows_v.at[pl.ds(0, 224)],
                                out_hbm.at[pl.ds(node_base + slab * 224, 224)])

            @pl.when(slab == 111)
            def _():
                pltpu.sync_copy(acc.at[pl.ds(slab * 224, 136)],
                                rows_v.at[pl.ds(0, 136)])
                pltpu.sync_copy(rows_v.at[pl.ds(0, 136)],
                                out_hbm.at[pl.ds(node_base + slab * 224, 136)])
            return carry

        lax.fori_loop(0, 7, wslab, 0)

    return s


_scatter48 = _make_scatter(48)
_scatter32 = _make_scatter(32)


def kernel(nfeat, atom_emb, efeat, A_lin1, A_lin2, e1_W, e1_b, e1_attn,
           n1_W, n1_b, e2_W, e2_b, e2_attn, e2_Wih, e2_Whh, n2_W, n2_b,
           n2_Wih, n2_Whh, e3_W, e3_b, e3_attn, e3_Wih, e3_Whh, n3_W, n3_b,
           n3_Wih, n3_Whh, out_W, out_b, edge_index):
    src = edge_index[0]
    dst = edge_index[1]
    src2 = jnp.pad(src, (0, EPAD - E)).reshape(EPAD // G_SUB, G_SUB)
    dst2 = jnp.pad(dst, (0, EPAD - E)).reshape(EPAD // G_SUB, G_SUB)
    zeros48 = jnp.zeros((224, 48), F32)
    zeros32 = jnp.zeros((224, 32), F32)
    ae98 = atom_emb.reshape(N, 98)

    h1 = _atom_call(nfeat, ae98, A_lin1.T, A_lin2.T)

    hs, hd = _gather96(h1, src2, dst2)
    e2_arr, exta, extb = _edge_call(hs, hd, efeat, None, e1_W, e1_b, e1_attn,
                                    None, None, None, None, 96, 92, 15)
    tsa = _scatter48(exta, dst, zeros48)
    tsb = _scatter32(extb, dst, zeros32)
    h2 = _node_call(h1, tsa, tsb, None, n1_W, n1_b, None, None, 96, 92)

    hs, hd = _gather64(h2, src2, dst2)
    e3_arr, ce, exta, extb = _edge_call(hs, hd, e2_arr, None, e2_W, e2_b,
                                        e2_attn, e2_Wih, e2_Whh, None, None,
                                        64, 64, 64)
    tsa = _scatter48(exta, dst, zeros48)
    tsb = _scatter32(extb, dst, zeros32)
    h3, hc = _node_call(h2, tsa, tsb, None, n2_W, n2_b, n2_Wih, n2_Whh, 64, 64)

    hs, hd = _gather64(h3, src2, dst2)
    e4_arr, ce, exta, extb = _edge_call(hs, hd, e3_arr, ce, e3_W, e3_b,
                                        e3_attn, e3_Wih, e3_Whh, None, None,
                                        64, 64, 64)
    tsa = _scatter48(exta, dst, zeros48)
    tsb = _scatter32(extb, dst, zeros32)
    h4, hc = _node_call(h3, tsa, tsb, hc, n3_W, n3_b, n3_Wih, n3_Whh, 64, 64)

    hs, hd = _gather64(h4, src2, dst2)
    out = _edge_call(hs, hd, e4_arr, ce, e3_W, e3_b, e3_attn,
                     e3_Wih, e3_Whh, out_W, out_b, 64, 64, 64)
    return out


# speedup vs baseline: 4.6823x; 1.0819x over previous
import functools
import jax
import jax.numpy as jnp
from jax import lax
from jax.experimental import pallas as pl
from jax.experimental.pallas import tpu as pltpu
from jax.experimental.pallas import tpu_sc as plsc

N = 50000
E = 800000
EPS = 1e-5
F32 = jnp.float32

EB = 3200
NB = 2000
EXTW = 80

G_SUB = 128
G_NSUB = 4
G_CHUNK = G_SUB * G_NSUB
EPAD = -(-E // G_CHUNK) * G_CHUNK
G_NCHUNK = EPAD // G_CHUNK
G_KMAX = (G_NCHUNK + 31) // 32

S_SUB = 128
S_NSUB = 4
S_CHUNK = S_SUB * S_NSUB
S_NCHUNK = EPAD // S_CHUNK
S_KMAX = (S_NCHUNK + 15) // 16
HALF = N // 2
ACC_ROWS = HALF + 8
ZROWS = 1568


def _rowspec(b, d):
    return pl.BlockSpec((b, d), lambda i: (i, 0))


def _full(shape):
    return pl.BlockSpec(shape, lambda i: tuple(0 for _ in shape))


def _dot(a, b):
    return jnp.dot(a, b, preferred_element_type=F32)


def _atom_body(nf_r, ae_r, a1t_r, a2t_r, out_r):
    ae = ae_r[...]
    y = None
    for a in range(14):
        xa = ae[:, 7 * a:7 * a + 7]
        m = jnp.mean(xa, axis=1, keepdims=True)
        v = jnp.mean(xa * xa, axis=1, keepdims=True) - m * m
        na = (xa - m) * lax.rsqrt(v + EPS)
        y = na if y is None else y + na
    y = y * (1.0 / 14.0)
    t = jnp.maximum(_dot(y, a1t_r[...]), 0.0)
    m2 = jnp.mean(t, axis=1, keepdims=True)
    v2 = jnp.mean(t * t, axis=1, keepdims=True) - m2 * m2
    t = (t - m2) * lax.rsqrt(v2 + EPS)
    t = jnp.maximum(_dot(t, a2t_r[...]), 0.0)
    z4 = jnp.zeros((t.shape[0], 4), F32)
    out_r[...] = jnp.concatenate([nf_r[...], t, z4], axis=1)


def _atom_call(nfeat, ae98, a1t, a2t):
    grid = (N // NB,)
    return pl.pallas_call(
        _atom_body,
        grid=grid,
        in_specs=[_rowspec(NB, 28), _rowspec(NB, 98), _full((7, 64)), _full((64, 64))],
        out_specs=_rowspec(NB, 96),
        out_shape=jax.ShapeDtypeStruct((N, 96), F32),
    )(nfeat, ae98, a1t, a2t)


def _make_edge_body(n_tot, lstm, has_c0, head):
    def body(*refs):
        i = 0
        hs_r = refs[i]; i += 1
        hd_r = refs[i]; i += 1
        e_r = refs[i]; i += 1
        c0_r = None
        if has_c0:
            c0_r = refs[i]; i += 1
        wst_r = refs[i]; i += 1
        wet_r = refs[i]; i += 1
        wdt_r = refs[i]; i += 1
        rs_r = refs[i]; i += 1
        b_r = refs[i]; i += 1
        wih_r = whh_r = None
        if lstm:
            wih_r = refs[i]; i += 1
            whh_r = refs[i]; i += 1
        if head:
            owt_r = refs[i]; i += 1
            ob_r = refs[i]; i += 1
            head_r = refs[i]; i += 1
        else:
            wa_r = refs[i]; i += 1
            eo_r = refs[i]; i += 1
            if lstm:
                co_r = refs[i]; i += 1
            exta_r = refs[i]; i += 1
            extb_r = refs[i]; i += 1

        hs = hs_r[...]
        hd = hd_r[...]
        e = e_r[...]
        s1 = (jnp.sum(hs, 1, keepdims=True) + jnp.sum(e, 1, keepdims=True)
              + jnp.sum(hd, 1, keepdims=True))
        s2 = (jnp.sum(hs * hs, 1, keepdims=True) + jnp.sum(e * e, 1, keepdims=True)
              + jnp.sum(hd * hd, 1, keepdims=True))
        m = s1 * (1.0 / n_tot)
        var = s2 * (1.0 / n_tot) - m * m
        rsig = lax.rsqrt(var + EPS)
        z = _dot(hs, wst_r[...]) + _dot(e, wet_r[...]) + _dot(hd, wdt_r[...])
        x = (z - m * rs_r[...]) * rsig + b_r[...]
        x = jnp.where(x >= 0, x, 0.01 * x)
        if lstm:
            gates = _dot(x, wih_r[...]) + _dot(e, whh_r[...])
            gi = jax.nn.sigmoid(gates[:, 0:64])
            gg = jnp.tanh(gates[:, 128:192])
            go = jax.nn.sigmoid(gates[:, 192:256])
            if has_c0:
                gf = jax.nn.sigmoid(gates[:, 64:128])
                c = gf * c0_r[...] + gi * gg
            else:
                c = gi * gg
            x = go * jnp.tanh(c)
        if head:
            head_r[...] = _dot(x, owt_r[...]) + ob_r[...]
            return
        eo_r[...] = x
        if lstm:
            co_r[...] = c
        attn = jnp.sum(x * wa_r[...], axis=1, keepdims=True)
        p = jnp.exp(attn)
        xp = x * p
        z15 = jnp.zeros((x.shape[0], 15), F32)
        exta_r[...] = xp[:, 0:48]
        extb_r[...] = jnp.concatenate([xp[:, 48:64], p, z15], axis=1)
    return body


def _edge_call(hs, hd, e, c0, W, b, wa, wih, whh, owt, ob, dh, dh_valid, de):
    lstm = wih is not None
    has_c0 = c0 is not None
    head = owt is not None
    n_tot = 2 * dh_valid + de
    wst = jnp.zeros((dh, 64), F32).at[:dh_valid].set(W[:, :dh_valid].T)
    wet = W[:, dh_valid:dh_valid + de].T
    wdt = jnp.zeros((dh, 64), F32).at[:dh_valid].set(W[:, dh_valid + de:].T)
    rs = W.sum(1)[None, :]
    b2 = b[None, :]
    grid = (E // EB,)
    args = [hs, hd, e]
    in_specs = [_rowspec(EB, dh), _rowspec(EB, dh), _rowspec(EB, de)]
    if has_c0:
        args.append(c0)
        in_specs.append(_rowspec(EB, 64))
    args += [wst, wet, wdt, rs, b2]
    in_specs += [_full((dh, 64)), _full((de, 64)), _full((dh, 64)),
                 _full((1, 64)), _full((1, 64))]
    if lstm:
        args += [wih.T, whh.T]
        in_specs += [_full((64, 256)), _full((64, 256))]
    if head:
        args += [owt.T, ob[None, :]]
        in_specs += [_full((64, 37)), _full((1, 37))]
        out_specs = _rowspec(EB, 37)
        out_shape = jax.ShapeDtypeStruct((E, 37), F32)
    else:
        args += [wa]
        in_specs += [_full((1, 64))]
        out_specs = [_rowspec(EB, 64)]
        out_shape = [jax.ShapeDtypeStruct((E, 64), F32)]
        if lstm:
            out_specs.append(_rowspec(EB, 64))
            out_shape.append(jax.ShapeDtypeStruct((E, 64), F32))
        out_specs.append(_rowspec(EB, 48))
        out_shape.append(jax.ShapeDtypeStruct((EPAD, 48), F32))
        out_specs.append(_rowspec(EB, 32))
        out_shape.append(jax.ShapeDtypeStruct((EPAD, 32), F32))
    body = _make_edge_body(n_tot, lstm, has_c0, head)
    return pl.pallas_call(
        body, grid=grid, in_specs=in_specs, out_specs=out_specs,
        out_shape=out_shape,
    )(*args)


def _make_node_body(n_tot, lstm, has_c0):
    def body(*refs):
        i = 0
        h_r = refs[i]; i += 1
        tsa_r = refs[i]; i += 1
        tsb_r = refs[i]; i += 1
        c0_r = None
        if has_c0:
            c0_r = refs[i]; i += 1
        wht_r = refs[i]; i += 1
        wat_r = refs[i]; i += 1
        rs_r = refs[i]; i += 1
        b_r = refs[i]; i += 1
        if lstm:
            wih_r = refs[i]; i += 1
            whh_r = refs[i]; i += 1
        ho_r = refs[i]; i += 1
        if lstm:
            co_r = refs[i]; i += 1

        h = h_r[...]
        tsa = tsa_r[...]
        tsb = tsb_r[...]
        s = tsb[:, 16:17]
        sinv = jnp.where(s > 0, 1.0 / s, 0.0)
        af = jnp.concatenate([tsa, tsb[:, 0:16]], axis=1) * sinv
        s1 = jnp.sum(h, 1, keepdims=True) + jnp.sum(af, 1, keepdims=True)
        s2 = jnp.sum(h * h, 1, keepdims=True) + jnp.sum(af * af, 1, keepdims=True)
        m = s1 * (1.0 / n_tot)
        var = s2 * (1.0 / n_tot) - m * m
        rsig = lax.rsqrt(var + EPS)
        z = _dot(h, wht_r[...]) + _dot(af, wat_r[...])
        x = (z - m * rs_r[...]) * rsig + b_r[...]
        x = jnp.maximum(x, 0.0)
        if lstm:
            gates = _dot(x, wih_r[...]) + _dot(h, whh_r[...])
            gi = jax.nn.sigmoid(gates[:, 0:64])
            gg = jnp.tanh(gates[:, 128:192])
            go = jax.nn.sigmoid(gates[:, 192:256])
            if has_c0:
                gf = jax.nn.sigmoid(gates[:, 64:128])
                c = gf * c0_r[...] + gi * gg
            else:
                c = gi * gg
            ho_r[...] = go * jnp.tanh(c)
            co_r[...] = c
        else:
            ho_r[...] = x
    return body


def _node_call(h, tsa, tsb, c0, W, b, wih, whh, dh, dh_valid):
    lstm = wih is not None
    has_c0 = c0 is not None
    n_tot = dh_valid + 64
    wht = jnp.zeros((dh, 64), F32).at[:dh_valid].set(W[:, :dh_valid].T)
    wat = W[:, dh_valid:].T
    rs = W.sum(1)[None, :]
    b2 = b[None, :]
    grid = (N // NB,)
    args = [h, tsa, tsb]
    in_specs = [_rowspec(NB, dh), _rowspec(NB, 48), _rowspec(NB, 32)]
    if has_c0:
        args.append(c0)
        in_specs.append(_rowspec(NB, 64))
    args += [wht, wat, rs, b2]
    in_specs += [_full((dh, 64)), _full((64, 64)), _full((1, 64)), _full((1, 64))]
    if lstm:
        args += [wih.T, whh.T]
        in_specs += [_full((64, 256)), _full((64, 256))]
        out_specs = [_rowspec(NB, 64), _rowspec(NB, 64)]
        out_shape = [jax.ShapeDtypeStruct((N, 64), F32),
                     jax.ShapeDtypeStruct((N, 64), F32)]
    else:
        out_specs = _rowspec(NB, 64)
        out_shape = jax.ShapeDtypeStruct((N, 64), F32)
    body = _make_node_body(n_tot, lstm, has_c0)
    return pl.pallas_call(
        body, grid=grid, in_specs=in_specs, out_specs=out_specs,
        out_shape=out_shape,
    )(*args)


def _make_gather(D):
    mesh = plsc.VectorSubcoreMesh(core_axis_name="c", subcore_axis_name="s")

    @functools.partial(
        pl.kernel, mesh=mesh,
        compiler_params=pltpu.CompilerParams(use_tc_tiling_on_sc=False),
        out_type=[jax.ShapeDtypeStruct((EPAD, D), F32),
                  jax.ShapeDtypeStruct((EPAD, D), F32)],
        scratch_types=[pltpu.VMEM((G_NSUB, G_SUB), jnp.int32),
                       pltpu.VMEM((G_NSUB, G_SUB), jnp.int32),
                       pltpu.VMEM((G_CHUNK, D), F32),
                       pltpu.VMEM((G_CHUNK, D), F32),
                       pltpu.SemaphoreType.DMA],
    )
    def g(h_hbm, src_hbm, dst_hbm, hs_hbm, hd_hbm, idxs_v, idxd_v,
          rows_s, rows_d, sem):
        wid = lax.axis_index("s") * 2 + lax.axis_index("c")

        def step(k, carry):
            j = wid + k * 32

            @pl.when(j < G_NCHUNK)
            def _():
                base = j * G_CHUNK
                c1 = pltpu.async_copy(src_hbm.at[pl.ds(j * G_NSUB, G_NSUB)],
                                      idxs_v, sem)
                c2 = pltpu.async_copy(dst_hbm.at[pl.ds(j * G_NSUB, G_NSUB)],
                                      idxd_v, sem)
                c1.wait()
                c2.wait()
                cps = [pltpu.async_copy(h_hbm.at[idxs_v.at[k2]],
                                        rows_s.at[pl.ds(k2 * G_SUB, G_SUB)],
                                        sem)
                       for k2 in range(G_NSUB)]
                cps += [pltpu.async_copy(h_hbm.at[idxd_v.at[k2]],
                                         rows_d.at[pl.ds(k2 * G_SUB, G_SUB)],
                                         sem)
                        for k2 in range(G_NSUB)]
                for cp in cps:
                    cp.wait()
                o1 = pltpu.async_copy(rows_s, hs_hbm.at[pl.ds(base, G_CHUNK)],
                                      sem)
                o2 = pltpu.async_copy(rows_d, hd_hbm.at[pl.ds(base, G_CHUNK)],
                                      sem)
                o1.wait()
                o2.wait()
            return carry

        lax.fori_loop(0, G_KMAX, step, 0)

    return g


_gather96 = _make_gather(96)
_gather64 = _make_gather(64)


def _make_scatter(W):
    mesh = plsc.VectorSubcoreMesh(core_axis_name="c", subcore_axis_name="s")

    @functools.partial(
        pl.kernel, mesh=mesh,
        compiler_params=pltpu.CompilerParams(use_tc_tiling_on_sc=False),
        out_type=jax.ShapeDtypeStruct((N, W), F32),
        scratch_types=[pltpu.VMEM((S_CHUNK,), jnp.int32),
                       pltpu.VMEM((S_SUB,), jnp.int32),
                       pltpu.VMEM((S_SUB,), jnp.int32),
                       pltpu.VMEM((S_SUB,), jnp.int32),
                       pltpu.VMEM((S_SUB,), jnp.int32),
                       pltpu.VMEM((S_CHUNK, W), F32),
                       pltpu.SemaphoreType.DMA,
                       pltpu.VMEM_SHARED((ACC_ROWS, W), F32)],
    )
    def s(ext_hbm, dst1_hbm, zeros_hbm, out_hbm, idx1_v, i2a, i2b, i2c, i2d,
          rows_v, sem, acc):
        cid = lax.axis_index("c")
        sid = lax.axis_index("s")
        node_base = cid * HALF

        pltpu.sync_copy(zeros_hbm, rows_v.at[pl.ds(0, 224)])

        def zslab(r, carry):
            slab = sid + r * 16

            @pl.when(slab < 111)
            def _():
                pltpu.sync_copy(rows_v.at[pl.ds(0, 224)],
                                acc.at[pl.ds(slab * 224, 224)])

            @pl.when(slab == 111)
            def _():
                pltpu.sync_copy(rows_v.at[pl.ds(0, 144)],
                                acc.at[pl.ds(slab * 224, 144)])
            return carry

        lax.fori_loop(0, 7, zslab, 0)
        plsc.subcore_barrier()

        idx_refs = [i2a, i2b, i2c, i2d]

        def step(k, carry):
            j = sid + k * 16

            @pl.when(j < S_NCHUNK)
            def _():
                ebase = j * S_CHUNK
                c1 = pltpu.async_copy(dst1_hbm.at[pl.ds(ebase, S_CHUNK)],
                                      idx1_v, sem)
                c2 = pltpu.async_copy(ext_hbm.at[pl.ds(ebase, S_CHUNK)],
                                      rows_v, sem)
                c1.wait()
                c2.wait()
                for g in range(S_CHUNK // 16):
                    d = idx1_v[pl.ds(g * 16, 16)]
                    local = d - node_base
                    ok = (local >= 0) & (local < HALF)
                    trash = HALF + (d & 7)
                    idx2 = jnp.where(ok, local, trash)
                    tgt = idx_refs[g // 8]
                    tgt[pl.ds((g % 8) * 16, 16)] = idx2
                for r in range(S_NSUB):
                    pltpu.sync_copy(rows_v.at[pl.ds(r * S_SUB, S_SUB)],
                                    acc.at[idx_refs[r]], add=True)
            return carry

        lax.fori_loop(0, S_KMAX, step, 0)
        plsc.subcore_barrier()

        def wslab(r, carry):
            slab = sid + r * 16

            @pl.when(slab < 111)
            def _():
                pltpu.sync_copy(acc.at[pl.ds(slab * 224, 224)],
                                rows_v.at[pl.ds(0, 224)])
                pltpu.sync_copy(rows_v.at[pl.ds(0, 224)],
                                out_hbm.at[pl.ds(node_base + slab * 224, 224)])

            @pl.when(slab == 111)
            def _():
                pltpu.sync_copy(acc.at[pl.ds(slab * 224, 136)],
                                rows_v.at[pl.ds(0, 136)])
                pltpu.sync_copy(rows_v.at[pl.ds(0, 136)],
                                out_hbm.at[pl.ds(node_base + slab * 224, 136)])
            return carry

        lax.fori_loop(0, 7, wslab, 0)

    return s


_scatter48 = _make_scatter(48)
_scatter32 = _make_scatter(32)


def kernel(nfeat, atom_emb, efeat, A_lin1, A_lin2, e1_W, e1_b, e1_attn,
           n1_W, n1_b, e2_W, e2_b, e2_attn, e2_Wih, e2_Whh, n2_W, n2_b,
           n2_Wih, n2_Whh, e3_W, e3_b, e3_attn, e3_Wih, e3_Whh, n3_W, n3_b,
           n3_Wih, n3_Whh, out_W, out_b, edge_index):
    src = edge_index[0]
    dst = edge_index[1]
    src2 = jnp.pad(src, (0, EPAD - E)).reshape(EPAD // G_SUB, G_SUB)
    dst2 = jnp.pad(dst, (0, EPAD - E)).reshape(EPAD // G_SUB, G_SUB)
    dstp = jnp.pad(dst, (0, EPAD - E), constant_values=-1)
    zeros48 = jnp.zeros((224, 48), F32)
    zeros32 = jnp.zeros((224, 32), F32)
    ae98 = atom_emb.reshape(N, 98)

    h1 = _atom_call(nfeat, ae98, A_lin1.T, A_lin2.T)

    hs, hd = _gather96(h1, src2, dst2)
    e2_arr, exta, extb = _edge_call(hs, hd, efeat, None, e1_W, e1_b, e1_attn,
                                    None, None, None, None, 96, 92, 15)
    tsa = _scatter48(exta, dstp, zeros48)
    tsb = _scatter32(extb, dstp, zeros32)
    h2 = _node_call(h1, tsa, tsb, None, n1_W, n1_b, None, None, 96, 92)

    hs, hd = _gather64(h2, src2, dst2)
    e3_arr, ce, exta, extb = _edge_call(hs, hd, e2_arr, None, e2_W, e2_b,
                                        e2_attn, e2_Wih, e2_Whh, None, None,
                                        64, 64, 64)
    tsa = _scatter48(exta, dstp, zeros48)
    tsb = _scatter32(extb, dstp, zeros32)
    h3, hc = _node_call(h2, tsa, tsb, None, n2_W, n2_b, n2_Wih, n2_Whh, 64, 64)

    hs, hd = _gather64(h3, src2, dst2)
    e4_arr, ce, exta, extb = _edge_call(hs, hd, e3_arr, ce, e3_W, e3_b,
                                        e3_attn, e3_Wih, e3_Whh, None, None,
                                        64, 64, 64)
    tsa = _scatter48(exta, dstp, zeros48)
    tsb = _scatter32(extb, dstp, zeros32)
    h4, hc = _node_call(h3, tsa, tsb, hc, n3_W, n3_b, n3_Wih, n3_Whh, 64, 64)

    hs, hd = _gather64(h4, src2, dst2)
    out = _edge_call(hs, hd, e4_arr, ce, e3_W, e3_b, e3_attn,
                     e3_Wih, e3_Whh, out_W, out_b, 64, 64, 64)
    return out


# speedup vs baseline: 5.0379x; 1.0760x over previous
import functools
import jax
import jax.numpy as jnp
from jax import lax
from jax.experimental import pallas as pl
from jax.experimental.pallas import tpu as pltpu
from jax.experimental.pallas import tpu_sc as plsc

N = 50000
E = 800000
EPS = 1e-5
F32 = jnp.float32

EB = 3200
NB = 2000
EXTW = 80

G_SUB = 128
G_NSUB = 4
G_CHUNK = G_SUB * G_NSUB
EPAD = -(-E // G_CHUNK) * G_CHUNK
G_NCHUNK = EPAD // G_CHUNK
G_KMAX = (G_NCHUNK + 31) // 32

S_SUB = 128
S_NSUB = 4
S_CHUNK = S_SUB * S_NSUB
S_NCHUNK = EPAD // S_CHUNK
S_KMAX = (S_NCHUNK + 15) // 16
HALF = N // 2
ACC_ROWS = HALF + 8
ZROWS = 1568


def _rowspec(b, d):
    return pl.BlockSpec((b, d), lambda i: (i, 0))


def _full(shape):
    return pl.BlockSpec(shape, lambda i: tuple(0 for _ in shape))


def _dot(a, b):
    return jnp.dot(a, b, preferred_element_type=F32)


def _atom_body(nf_r, ae_r, a1t_r, a2t_r, out_r):
    ae = ae_r[...]
    y = None
    for a in range(14):
        xa = ae[:, 7 * a:7 * a + 7]
        m = jnp.mean(xa, axis=1, keepdims=True)
        v = jnp.mean(xa * xa, axis=1, keepdims=True) - m * m
        na = (xa - m) * lax.rsqrt(v + EPS)
        y = na if y is None else y + na
    y = y * (1.0 / 14.0)
    t = jnp.maximum(_dot(y, a1t_r[...]), 0.0)
    m2 = jnp.mean(t, axis=1, keepdims=True)
    v2 = jnp.mean(t * t, axis=1, keepdims=True) - m2 * m2
    t = (t - m2) * lax.rsqrt(v2 + EPS)
    t = jnp.maximum(_dot(t, a2t_r[...]), 0.0)
    z4 = jnp.zeros((t.shape[0], 4), F32)
    out_r[...] = jnp.concatenate([nf_r[...], t, z4], axis=1)


def _atom_call(nfeat, ae98, a1t, a2t):
    grid = (N // NB,)
    return pl.pallas_call(
        _atom_body,
        grid=grid,
        in_specs=[_rowspec(NB, 28), _rowspec(NB, 98), _full((7, 64)), _full((64, 64))],
        out_specs=_rowspec(NB, 96),
        out_shape=jax.ShapeDtypeStruct((N, 96), F32),
    )(nfeat, ae98, a1t, a2t)


def _make_edge_body(n_tot, lstm, has_c0, head):
    def body(*refs):
        i = 0
        hs_r = refs[i]; i += 1
        hd_r = refs[i]; i += 1
        e_r = refs[i]; i += 1
        c0_r = None
        if has_c0:
            c0_r = refs[i]; i += 1
        wst_r = refs[i]; i += 1
        wet_r = refs[i]; i += 1
        wdt_r = refs[i]; i += 1
        rs_r = refs[i]; i += 1
        b_r = refs[i]; i += 1
        wih_r = whh_r = None
        if lstm:
            wih_r = refs[i]; i += 1
            whh_r = refs[i]; i += 1
        if head:
            owt_r = refs[i]; i += 1
            ob_r = refs[i]; i += 1
            head_r = refs[i]; i += 1
        else:
            wa_r = refs[i]; i += 1
            eo_r = refs[i]; i += 1
            if lstm:
                co_r = refs[i]; i += 1
            exta_r = refs[i]; i += 1
            extb_r = refs[i]; i += 1

        hs = hs_r[...]
        hd = hd_r[...]
        e = e_r[...]
        z65 = (_dot(hs, wst_r[...]) + _dot(e, wet_r[...])
               + _dot(hd, wdt_r[...]))
        z = z65[:, 0:64]
        s1 = z65[:, 64:65]
        s2 = z65[:, 65:66]
        sq65 = (_dot(hs * hs, wst_r[...]) + _dot(e * e, wet_r[...])
                + _dot(hd * hd, wdt_r[...]))
        s2 = sq65[:, 64:65]
        m = s1 * (1.0 / n_tot)
        var = s2 * (1.0 / n_tot) - m * m
        rsig = lax.rsqrt(var + EPS)
        x = (z - m * rs_r[...]) * rsig + b_r[...]
        x = jnp.where(x >= 0, x, 0.01 * x)
        if lstm:
            gates = _dot(x, wih_r[...]) + _dot(e, whh_r[...])
            gi = jax.nn.sigmoid(gates[:, 0:64])
            gg = jnp.tanh(gates[:, 128:192])
            go = jax.nn.sigmoid(gates[:, 192:256])
            if has_c0:
                gf = jax.nn.sigmoid(gates[:, 64:128])
                c = gf * c0_r[...] + gi * gg
            else:
                c = gi * gg
            x = go * jnp.tanh(c)
        if head:
            head_r[...] = _dot(x, owt_r[...]) + ob_r[...]
            return
        eo_r[...] = x
        if lstm:
            co_r[...] = c
        attn = _dot(x, wa_r[...])
        p = jnp.exp(attn[:, 0:1])
        xp = x * p
        z15 = jnp.zeros((x.shape[0], 15), F32)
        exta_r[...] = xp[:, 0:48]
        extb_r[...] = jnp.concatenate([xp[:, 48:64], p, z15], axis=1)
    return body


def _edge_call(hs, hd, e, c0, W, b, wa, wih, whh, owt, ob, dh, dh_valid, de):
    lstm = wih is not None
    has_c0 = c0 is not None
    head = owt is not None
    n_tot = 2 * dh_valid + de
    wst = jnp.zeros((dh, 72), F32).at[:dh_valid, 0:64].set(W[:, :dh_valid].T)
    wst = wst.at[:dh_valid, 64].set(1.0)
    wet = jnp.zeros((de, 72), F32).at[:, 0:64].set(W[:, dh_valid:dh_valid + de].T)
    wet = wet.at[:, 64].set(1.0)
    wdt = jnp.zeros((dh, 72), F32).at[:dh_valid, 0:64].set(W[:, dh_valid + de:].T)
    wdt = wdt.at[:dh_valid, 64].set(1.0)
    rs = W.sum(1)[None, :]
    b2 = b[None, :]
    grid = (E // EB,)
    args = [hs, hd, e]
    in_specs = [_rowspec(EB, dh), _rowspec(EB, dh), _rowspec(EB, de)]
    if has_c0:
        args.append(c0)
        in_specs.append(_rowspec(EB, 64))
    args += [wst, wet, wdt, rs, b2]
    in_specs += [_full((dh, 72)), _full((de, 72)), _full((dh, 72)),
                 _full((1, 64)), _full((1, 64))]
    if lstm:
        args += [wih.T, whh.T]
        in_specs += [_full((64, 256)), _full((64, 256))]
    if head:
        args += [owt.T, ob[None, :]]
        in_specs += [_full((64, 37)), _full((1, 37))]
        out_specs = _rowspec(EB, 37)
        out_shape = jax.ShapeDtypeStruct((E, 37), F32)
    else:
        args += [jnp.zeros((64, 8), F32).at[:, 0].set(wa[0])]
        in_specs += [_full((64, 8))]
        out_specs = [_rowspec(EB, 64)]
        out_shape = [jax.ShapeDtypeStruct((E, 64), F32)]
        if lstm:
            out_specs.append(_rowspec(EB, 64))
            out_shape.append(jax.ShapeDtypeStruct((E, 64), F32))
        out_specs.append(_rowspec(EB, 48))
        out_shape.append(jax.ShapeDtypeStruct((EPAD, 48), F32))
        out_specs.append(_rowspec(EB, 32))
        out_shape.append(jax.ShapeDtypeStruct((EPAD, 32), F32))
    body = _make_edge_body(n_tot, lstm, has_c0, head)
    return pl.pallas_call(
        body, grid=grid, in_specs=in_specs, out_specs=out_specs,
        out_shape=out_shape,
    )(*args)


def _make_node_body(n_tot, lstm, has_c0):
    def body(*refs):
        i = 0
        h_r = refs[i]; i += 1
        tsa_r = refs[i]; i += 1
        tsb_r = refs[i]; i += 1
        c0_r = None
        if has_c0:
            c0_r = refs[i]; i += 1
        wht_r = refs[i]; i += 1
        wat_r = refs[i]; i += 1
        rs_r = refs[i]; i += 1
        b_r = refs[i]; i += 1
        if lstm:
            wih_r = refs[i]; i += 1
            whh_r = refs[i]; i += 1
        ho_r = refs[i]; i += 1
        if lstm:
            co_r = refs[i]; i += 1

        h = h_r[...]
        tsa = tsa_r[...]
        tsb = tsb_r[...]
        s = tsb[:, 16:17]
        sinv = jnp.where(s > 0, 1.0 / s, 0.0)
        af = jnp.concatenate([tsa, tsb[:, 0:16]], axis=1) * sinv
        z65 = _dot(h, wht_r[...]) + _dot(af, wat_r[...])
        z = z65[:, 0:64]
        s1 = z65[:, 64:65]
        sq65 = _dot(h * h, wht_r[...]) + _dot(af * af, wat_r[...])
        s2 = sq65[:, 64:65]
        m = s1 * (1.0 / n_tot)
        var = s2 * (1.0 / n_tot) - m * m
        rsig = lax.rsqrt(var + EPS)
        x = (z - m * rs_r[...]) * rsig + b_r[...]
        x = jnp.maximum(x, 0.0)
        if lstm:
            gates = _dot(x, wih_r[...]) + _dot(h, whh_r[...])
            gi = jax.nn.sigmoid(gates[:, 0:64])
            gg = jnp.tanh(gates[:, 128:192])
            go = jax.nn.sigmoid(gates[:, 192:256])
            if has_c0:
                gf = jax.nn.sigmoid(gates[:, 64:128])
                c = gf * c0_r[...] + gi * gg
            else:
                c = gi * gg
            ho_r[...] = go * jnp.tanh(c)
            co_r[...] = c
        else:
            ho_r[...] = x
    return body


def _node_call(h, tsa, tsb, c0, W, b, wih, whh, dh, dh_valid):
    lstm = wih is not None
    has_c0 = c0 is not None
    n_tot = dh_valid + 64
    wht = jnp.zeros((dh, 72), F32).at[:dh_valid, 0:64].set(W[:, :dh_valid].T)
    wht = wht.at[:dh_valid, 64].set(1.0)
    wat = jnp.zeros((64, 72), F32).at[:, 0:64].set(W[:, dh_valid:].T)
    wat = wat.at[:, 64].set(1.0)
    rs = W.sum(1)[None, :]
    b2 = b[None, :]
    grid = (N // NB,)
    args = [h, tsa, tsb]
    in_specs = [_rowspec(NB, dh), _rowspec(NB, 48), _rowspec(NB, 32)]
    if has_c0:
        args.append(c0)
        in_specs.append(_rowspec(NB, 64))
    args += [wht, wat, rs, b2]
    in_specs += [_full((dh, 72)), _full((64, 72)), _full((1, 64)), _full((1, 64))]
    if lstm:
        args += [wih.T, whh.T]
        in_specs += [_full((64, 256)), _full((64, 256))]
        out_specs = [_rowspec(NB, 64), _rowspec(NB, 64)]
        out_shape = [jax.ShapeDtypeStruct((N, 64), F32),
                     jax.ShapeDtypeStruct((N, 64), F32)]
    else:
        out_specs = _rowspec(NB, 64)
        out_shape = jax.ShapeDtypeStruct((N, 64), F32)
    body = _make_node_body(n_tot, lstm, has_c0)
    return pl.pallas_call(
        body, grid=grid, in_specs=in_specs, out_specs=out_specs,
        out_shape=out_shape,
    )(*args)


def _make_gather(D):
    mesh = plsc.VectorSubcoreMesh(core_axis_name="c", subcore_axis_name="s")

    @functools.partial(
        pl.kernel, mesh=mesh,
        compiler_params=pltpu.CompilerParams(use_tc_tiling_on_sc=False),
        out_type=[jax.ShapeDtypeStruct((EPAD, D), F32),
                  jax.ShapeDtypeStruct((EPAD, D), F32)],
        scratch_types=[pltpu.VMEM((G_NSUB, G_SUB), jnp.int32),
                       pltpu.VMEM((G_NSUB, G_SUB), jnp.int32),
                       pltpu.VMEM((G_CHUNK, D), F32),
                       pltpu.VMEM((G_CHUNK, D), F32),
                       pltpu.SemaphoreType.DMA],
    )
    def g(h_hbm, src_hbm, dst_hbm, hs_hbm, hd_hbm, idxs_v, idxd_v,
          rows_s, rows_d, sem):
        wid = lax.axis_index("s") * 2 + lax.axis_index("c")

        def step(k, carry):
            j = wid + k * 32

            @pl.when(j < G_NCHUNK)
            def _():
                base = j * G_CHUNK
                c1 = pltpu.async_copy(src_hbm.at[pl.ds(j * G_NSUB, G_NSUB)],
                                      idxs_v, sem)
                c2 = pltpu.async_copy(dst_hbm.at[pl.ds(j * G_NSUB, G_NSUB)],
                                      idxd_v, sem)
                c1.wait()
                c2.wait()
                cps = [pltpu.async_copy(h_hbm.at[idxs_v.at[k2]],
                                        rows_s.at[pl.ds(k2 * G_SUB, G_SUB)],
                                        sem)
                       for k2 in range(G_NSUB)]
                cps += [pltpu.async_copy(h_hbm.at[idxd_v.at[k2]],
                                         rows_d.at[pl.ds(k2 * G_SUB, G_SUB)],
                                         sem)
                        for k2 in range(G_NSUB)]
                for cp in cps:
                    cp.wait()
                o1 = pltpu.async_copy(rows_s, hs_hbm.at[pl.ds(base, G_CHUNK)],
                                      sem)
                o2 = pltpu.async_copy(rows_d, hd_hbm.at[pl.ds(base, G_CHUNK)],
                                      sem)
                o1.wait()
                o2.wait()
            return carry

        lax.fori_loop(0, G_KMAX, step, 0)

    return g


_gather96 = _make_gather(96)
_gather64 = _make_gather(64)


def _make_scatter(W):
    mesh = plsc.VectorSubcoreMesh(core_axis_name="c", subcore_axis_name="s")

    @functools.partial(
        pl.kernel, mesh=mesh,
        compiler_params=pltpu.CompilerParams(use_tc_tiling_on_sc=False),
        out_type=jax.ShapeDtypeStruct((N, W), F32),
        scratch_types=[pltpu.VMEM((S_CHUNK,), jnp.int32),
                       pltpu.VMEM((S_SUB,), jnp.int32),
                       pltpu.VMEM((S_SUB,), jnp.int32),
                       pltpu.VMEM((S_SUB,), jnp.int32),
                       pltpu.VMEM((S_SUB,), jnp.int32),
                       pltpu.VMEM((S_CHUNK, W), F32),
                       pltpu.SemaphoreType.DMA,
                       pltpu.VMEM_SHARED((ACC_ROWS, W), F32)],
    )
    def s(ext_hbm, dst1_hbm, zeros_hbm, out_hbm, idx1_v, i2a, i2b, i2c, i2d,
          rows_v, sem, acc):
        cid = lax.axis_index("c")
        sid = lax.axis_index("s")
        node_base = cid * HALF

        pltpu.sync_copy(zeros_hbm, rows_v.at[pl.ds(0, 224)])

        def zslab(r, carry):
            slab = sid + r * 16

            @pl.when(slab < 111)
            def _():
                pltpu.sync_copy(rows_v.at[pl.ds(0, 224)],
                                acc.at[pl.ds(slab * 224, 224)])

            @pl.when(slab == 111)
            def _():
                pltpu.sync_copy(rows_v.at[pl.ds(0, 144)],
                                acc.at[pl.ds(slab * 224, 144)])
            return carry

        lax.fori_loop(0, 7, zslab, 0)
        plsc.subcore_barrier()

        idx_refs = [i2a, i2b, i2c, i2d]

        def step(k, carry):
            j = sid + k * 16

            @pl.when(j < S_NCHUNK)
            def _():
                ebase = j * S_CHUNK
                c1 = pltpu.async_copy(dst1_hbm.at[pl.ds(ebase, S_CHUNK)],
                                      idx1_v, sem)
                c2 = pltpu.async_copy(ext_hbm.at[pl.ds(ebase, S_CHUNK)],
                                      rows_v, sem)
                c1.wait()
                c2.wait()
                for g in range(S_CHUNK // 16):
                    d = idx1_v[pl.ds(g * 16, 16)]
                    local = d - node_base
                    ok = (local >= 0) & (local < HALF)
                    trash = HALF + (d & 7)
                    idx2 = jnp.where(ok, local, trash)
                    tgt = idx_refs[g // 8]
                    tgt[pl.ds((g % 8) * 16, 16)] = idx2
                for r in range(S_NSUB):
                    pltpu.sync_copy(rows_v.at[pl.ds(r * S_SUB, S_SUB)],
                                    acc.at[idx_refs[r]], add=True)
            return carry

        lax.fori_loop(0, S_KMAX, step, 0)
        plsc.subcore_barrier()

        def wslab(r, carry):
            slab = sid + r * 16

            @pl.when(slab < 111)
            def _():
                pltpu.sync_copy(acc.at[pl.ds(slab * 224, 224)],
                                rows_v.at[pl.ds(0, 224)])
                pltpu.sync_copy(rows_v.at[pl.ds(0, 224)],
                                out_hbm.at[pl.ds(node_base + slab * 224, 224)])

            @pl.when(slab == 111)
            def _():
                pltpu.sync_copy(acc.at[pl.ds(slab * 224, 136)],
                                rows_v.at[pl.ds(0, 136)])
                pltpu.sync_copy(rows_v.at[pl.ds(0, 136)],
                                out_hbm.at[pl.ds(node_base + slab * 224, 136)])
            return carry

        lax.fori_loop(0, 7, wslab, 0)

    return s


_scatter48 = _make_scatter(48)
_scatter32 = _make_scatter(32)


def kernel(nfeat, atom_emb, efeat, A_lin1, A_lin2, e1_W, e1_b, e1_attn,
           n1_W, n1_b, e2_W, e2_b, e2_attn, e2_Wih, e2_Whh, n2_W, n2_b,
           n2_Wih, n2_Whh, e3_W, e3_b, e3_attn, e3_Wih, e3_Whh, n3_W, n3_b,
           n3_Wih, n3_Whh, out_W, out_b, edge_index):
    src = edge_index[0]
    dst = edge_index[1]
    src2 = jnp.pad(src, (0, EPAD - E)).reshape(EPAD // G_SUB, G_SUB)
    dst2 = jnp.pad(dst, (0, EPAD - E)).reshape(EPAD // G_SUB, G_SUB)
    dstp = jnp.pad(dst, (0, EPAD - E), constant_values=-1)
    zeros48 = jnp.zeros((224, 48), F32)
    zeros32 = jnp.zeros((224, 32), F32)
    ae98 = atom_emb.reshape(N, 98)

    h1 = _atom_call(nfeat, ae98, A_lin1.T, A_lin2.T)

    hs, hd = _gather96(h1, src2, dst2)
    e2_arr, exta, extb = _edge_call(hs, hd, efeat, None, e1_W, e1_b, e1_attn,
                                    None, None, None, None, 96, 92, 15)
    tsa = _scatter48(exta, dstp, zeros48)
    tsb = _scatter32(extb, dstp, zeros32)
    h2 = _node_call(h1, tsa, tsb, None, n1_W, n1_b, None, None, 96, 92)

    hs, hd = _gather64(h2, src2, dst2)
    e3_arr, ce, exta, extb = _edge_call(hs, hd, e2_arr, None, e2_W, e2_b,
                                        e2_attn, e2_Wih, e2_Whh, None, None,
                                        64, 64, 64)
    tsa = _scatter48(exta, dstp, zeros48)
    tsb = _scatter32(extb, dstp, zeros32)
    h3, hc = _node_call(h2, tsa, tsb, None, n2_W, n2_b, n2_Wih, n2_Whh, 64, 64)

    hs, hd = _gather64(h3, src2, dst2)
    e4_arr, ce, exta, extb = _edge_call(hs, hd, e3_arr, ce, e3_W, e3_b,
                                        e3_attn, e3_Wih, e3_Whh, None, None,
                                        64, 64, 64)
    tsa = _scatter48(exta, dstp, zeros48)
    tsb = _scatter32(extb, dstp, zeros32)
    h4, hc = _node_call(h3, tsa, tsb, hc, n3_W, n3_b, n3_Wih, n3_Whh, 64, 64)

    hs, hd = _gather64(h4, src2, dst2)
    out = _edge_call(hs, hd, e4_arr, ce, e3_W, e3_b, e3_attn,
                     e3_Wih, e3_Whh, out_W, out_b, 64, 64, 64)
    return out
